# baseline (speedup 1.0000x reference)
import functools

import jax
import jax.numpy as jnp
from jax import lax
from jax.experimental import pallas as pl
from jax.experimental.pallas import tpu as pltpu
from jax.experimental.pallas import tpu_sc as plsc

N = 10000
D = 128
H = 64
C = 16
E = 320000

NC = 2
NS = 16
NW = NC * NS
CH = 128
NB = 5
K = -(-E // (NW * CH * 2 * NB)) * 2 * NB
E_PAD = NW * CH * K
K2 = 2 * K
HW = H // NC
NPAD = 10240
ZPT = NPAD // NS

_MESH = plsc.VectorSubcoreMesh(core_axis_name="c", subcore_axis_name="s")


@functools.partial(
    pl.kernel,
    out_type=jax.ShapeDtypeStruct((NC * NPAD,), jnp.float32),
    mesh=_MESH,
    scratch_types=[
        pltpu.VMEM((K, CH), jnp.int32),
        pltpu.VMEM((CH,), jnp.float32),
        pltpu.VMEM((ZPT,), jnp.float32),
        pltpu.VMEM_SHARED((NPAD,), jnp.float32),
        pltpu.SemaphoreType.DMA((2 * NB,)),
    ],
)
def _deg_kernel(dst_hbm, deg_out, dst_v, ones_v, stage_v, deg_sh, sems):
    c = lax.axis_index("c")
    s = lax.axis_index("s")
    wid = c * NS + s
    nsem = 2 * NB

    def fill(buf, n, value):
        def fbody(j, carry):
            buf[pl.ds(j * 16, 16)] = jnp.full((16,), value, jnp.float32)
            return carry

        lax.fori_loop(0, n // 16, fbody, 0)

    fill(ones_v, CH, 1.0)
    fill(stage_v, ZPT, 0.0)
    pltpu.sync_copy(stage_v, deg_sh.at[pl.ds(s * ZPT, ZPT)])
    pltpu.sync_copy(dst_hbm.at[wid], dst_v)
    plsc.subcore_barrier()

    def sc_start(i, j):
        pltpu.async_copy(ones_v, deg_sh.at[dst_v.at[j]], sems.at[i], add=True)

    def sc_wait(i, j):
        pltpu.make_async_copy(ones_v, deg_sh.at[dst_v.at[j]], sems.at[i]).wait()

    for b in range(nsem):
        sc_start(b, b)

    def body(t, carry):
        base = t * nsem
        for b in range(nsem):
            sc_wait(b, base + b)
            sc_start(b, base + nsem + b)
        return carry

    lax.fori_loop(0, K // nsem - 1, body, 0)
    for b in range(nsem):
        sc_wait(b, K - nsem + b)
    plsc.subcore_barrier()
    pltpu.sync_copy(deg_sh.at[pl.ds(s * ZPT, ZPT)], stage_v)
    pltpu.sync_copy(stage_v, deg_out.at[pl.ds(c * NPAD + s * ZPT, ZPT)])


def _rsqrt16(x):
    i = plsc.bitcast(x, jnp.int32)
    i = jnp.int32(0x5F3759DF) - lax.shift_right_arithmetic(i, 1)
    y = plsc.bitcast(i, jnp.float32)
    for _ in range(3):
        y = y * (1.5 - 0.5 * x * y * y)
    return y


def _make_agg_kernel(width, col_split):
    nk = K2 if col_split else K
    kh = 50 if col_split else 20

    @functools.partial(
        pl.kernel,
        out_type=[
            jax.ShapeDtypeStruct((NC, NPAD, width), jnp.float32),
            jax.ShapeDtypeStruct((NC, NPAD, width), jnp.float32),
        ],
        mesh=_MESH,
        scratch_types=[
            pltpu.VMEM((nk, CH), jnp.int32),
            pltpu.VMEM((nk, CH), jnp.int32),
            pltpu.VMEM((2 * NB, CH, width), jnp.float32),
            pltpu.VMEM((ZPT,), jnp.float32),
            pltpu.VMEM((ZPT,), jnp.float32),
            pltpu.VMEM_SHARED((NPAD, width), jnp.float32),
            pltpu.VMEM_SHARED((NPAD, width), jnp.float32),
            pltpu.SemaphoreType.DMA((2 * NB,)),
        ],
        compiler_params=pltpu.CompilerParams(
            use_tc_tiling_on_sc=False, needs_layout_passes=False
        ),
    )
    def _agg(src_hbm, dst_hbm, y_hbm, deg_hbm, zeros_hbm, out_hbm, ysc_hbm,
             src_v, dst_v, bufs, dinv_v, dtmp_v, agg_sh, y_sh, sems):
        c = lax.axis_index("c")
        s = lax.axis_index("s")
        idx_row = s if col_split else c * NS + s
        iota16 = lax.iota(jnp.int32, 16)

        def scale_buf(buf, rb):
            def rbody(rr, carry):
                for u in range(2):
                    r = rr * 2 + u
                    d16 = plsc.load_gather(dinv_v, [jnp.full((16,), rb + r, jnp.int32)])
                    for cc in range(width // 16):
                        sl = pl.ds(cc * 16, 16)
                        buf[r, sl] = buf[r, sl] * d16
                return carry

            lax.fori_loop(0, CH // 2, rbody, 0)

        def scatter_start(i, j):
            pltpu.async_copy(bufs.at[i], agg_sh.at[dst_v.at[j]], sems.at[i], add=True)

        def scatter_wait(i, j):
            pltpu.make_async_copy(bufs.at[i], agg_sh.at[dst_v.at[j]], sems.at[i]).wait()

        nst = ZPT // CH

        def ysl(i):
            return pl.ds(s * ZPT + i * CH, CH)

        def ysrc(i):
            if col_split:
                return y_hbm.at[c, ysl(i)]
            return y_hbm.at[ysl(i)]

        pltpu.async_copy(src_hbm.at[idx_row], src_v, sems.at[6])
        pltpu.async_copy(dst_hbm.at[idx_row], dst_v, sems.at[7])
        pltpu.sync_copy(deg_hbm.at[pl.ds(s * ZPT, ZPT)], dinv_v)
        pltpu.sync_copy(deg_hbm.at[pl.ds(NPAD + s * ZPT, ZPT)], dtmp_v)

        def dbody(j, carry):
            dsl = pl.ds(j * 16, 16)
            deg = dinv_v[dsl] + dtmp_v[dsl] + 1.0
            dinv_v[dsl] = _rsqrt16(deg)
            return carry

        lax.fori_loop(0, ZPT // 16, dbody, 0)

        pltpu.async_copy(ysrc(0), bufs.at[1], sems.at[1])
        for i in range(nst):
            b = 1 + (i % 2)
            pltpu.make_async_copy(ysrc(i), bufs.at[b], sems.at[b]).wait()
            if i + 1 < nst:
                nb = 1 + ((i + 1) % 2)
                if i >= 1:
                    pltpu.make_async_copy(
                        bufs.at[nb], ysc_hbm.at[c, ysl(i - 1)], sems.at[nb]
                    ).wait()
                pltpu.async_copy(ysrc(i + 1), bufs.at[nb], sems.at[nb])
            scale_buf(bufs.at[b], i * CH)
            pltpu.sync_copy(bufs.at[b], y_sh.at[ysl(i)])
            if col_split:
                pltpu.sync_copy(bufs.at[b], agg_sh.at[ysl(i)])
            else:
                @pl.when(c == 0)
                def _():
                    pltpu.sync_copy(bufs.at[b], agg_sh.at[ysl(i)])
            pltpu.async_copy(bufs.at[b], ysc_hbm.at[c, ysl(i)], sems.at[b])
        for i in (nst - 2, nst - 1):
            b = 1 + (i % 2)
            pltpu.make_async_copy(bufs.at[b], ysc_hbm.at[c, ysl(i)], sems.at[b]).wait()
        if not col_split:
            @pl.when(c == 1)
            def _():
                pltpu.sync_copy(zeros_hbm, bufs.at[0])
                for i in range(nst):
                    pltpu.sync_copy(bufs.at[0], agg_sh.at[ysl(i)])
        pltpu.make_async_copy(src_hbm.at[idx_row], src_v, sems.at[6]).wait()
        pltpu.make_async_copy(dst_hbm.at[idx_row], dst_v, sems.at[7]).wait()
        plsc.subcore_barrier()

        def run_pipeline(lo, hi, ysrc_ref):
            def gstart(i, j):
                pltpu.async_copy(ysrc_ref.at[src_v.at[j]], bufs.at[i], sems.at[i])

            def gwait(i, j):
                pltpu.make_async_copy(ysrc_ref.at[src_v.at[j]], bufs.at[i], sems.at[i]).wait()

            for g in range(2):
                for b in range(NB):
                    gstart(g * NB + b, lo + g * NB + b)

            def body(tt, carry):
                for g in range(2):
                    base = lo + (2 * tt + g) * NB
                    for b in range(NB):
                        gwait(g * NB + b, base + b)
                        scatter_start(g * NB + b, base + b)
                    nbase = base + 2 * NB

                    @pl.when(nbase < hi)
                    def _():
                        for b in range(NB):
                            scatter_wait(g * NB + b, base + b)
                            gstart(g * NB + b, nbase + b)

                return carry

            lax.fori_loop(0, (hi - lo) // (2 * NB), body, 0)
            for g in range(2):
                base = hi - 2 * NB + g * NB
                for b in range(NB):
                    scatter_wait(g * NB + b, base + b)

        run_pipeline(0, kh, ysc_hbm.at[c])
        run_pipeline(kh, nk, y_sh)
        plsc.subcore_barrier()
        for i in range(nst):
            b = i % 2
            if i >= 2:
                pltpu.make_async_copy(
                    bufs.at[b], out_hbm.at[c, ysl(i - 2)], sems.at[b]
                ).wait()
            pltpu.sync_copy(agg_sh.at[ysl(i)], bufs.at[b])
            scale_buf(bufs.at[b], i * CH)
            pltpu.async_copy(bufs.at[b], out_hbm.at[c, ysl(i)], sems.at[b])
        for i in range(nst - 2, nst):
            pltpu.make_async_copy(bufs.at[i % 2], out_hbm.at[c, ysl(i)], sems.at[i % 2]).wait()

    return _agg


_agg_l1 = _make_agg_kernel(HW, col_split=True)
_agg_l2 = _make_agg_kernel(C, col_split=False)

_RB = 1024
_GRID = NPAD // _RB
_RB0 = 1000


def _tca0_body(x_ref, w1_ref, xw_ref):
    xw = jnp.dot(x_ref[...], w1_ref[...], preferred_element_type=jnp.float32)
    xw_ref[0] = xw[:, :HW]
    xw_ref[1] = xw[:, HW:]


def _tca0(x, W1):
    return pl.pallas_call(
        _tca0_body,
        grid=(N // _RB0,),
        in_specs=[
            pl.BlockSpec((_RB0, D), lambda i: (i, 0)),
            pl.BlockSpec((D, H), lambda i: (0, 0)),
        ],
        out_specs=pl.BlockSpec((NC, _RB0, HW), lambda i: (0, i, 0)),
        out_shape=jax.ShapeDtypeStruct((NC, NPAD, HW), jnp.float32),
    )(x, W1)


def _tcb_body(agg_ref, b1_ref, w2_ref, y2_ref):
    pre = jnp.concatenate([agg_ref[0], agg_ref[1]], axis=1) + b1_ref[...]
    h = jnp.maximum(pre, 0.0)
    y2_ref[...] = jnp.dot(h, w2_ref[...], preferred_element_type=jnp.float32)


def _tcb(agg1, b1, W2):
    return pl.pallas_call(
        _tcb_body,
        grid=(_GRID,),
        in_specs=[
            pl.BlockSpec((NC, _RB, HW), lambda i: (0, i, 0)),
            pl.BlockSpec((1, H), lambda i: (0, 0)),
            pl.BlockSpec((H, C), lambda i: (0, 0)),
        ],
        out_specs=pl.BlockSpec((_RB, C), lambda i: (i, 0)),
        out_shape=jax.ShapeDtypeStruct((NPAD, C), jnp.float32),
    )(agg1, b1, W2)


def _tcc_body(agg_ref, b2_ref, out_ref):
    o = agg_ref[0] + agg_ref[1] + b2_ref[...]
    m = jnp.max(o, axis=1, keepdims=True)
    lse = jnp.log(jnp.sum(jnp.exp(o - m), axis=1, keepdims=True)) + m
    out_ref[...] = o - lse


def _tcc(agg2, b2):
    return pl.pallas_call(
        _tcc_body,
        grid=(_GRID,),
        in_specs=[
            pl.BlockSpec((NC, _RB, C), lambda i: (0, i, 0)),
            pl.BlockSpec((1, C), lambda i: (0, 0)),
        ],
        out_specs=pl.BlockSpec((_RB, C), lambda i: (i, 0)),
        out_shape=jax.ShapeDtypeStruct((NPAD, C), jnp.float32),
    )(agg2, b2)


def kernel(x, edge_index, W1, b1, W2, b2):
    src = edge_index[0]
    dst = edge_index[1]
    pad = E_PAD - E
    srcp = jnp.concatenate([src, jnp.full((pad,), N, jnp.int32)])
    dstp = jnp.concatenate([dst, jnp.full((pad,), N, jnp.int32)])
    src32 = srcp.reshape(NW, K, CH)
    dst32 = dstp.reshape(NW, K, CH)
    src16 = srcp.reshape(NS, K2, CH)
    dst16 = dstp.reshape(NS, K2, CH)
    z32 = jnp.zeros((CH, HW), jnp.float32)
    z16 = jnp.zeros((CH, C), jnp.float32)

    xw = _tca0(x, W1)
    deg = _deg_kernel(dst32)
    agg1, _y1 = _agg_l1(src16, dst16, xw, deg, z32)
    y2 = _tcb(agg1, b1.reshape(1, H), W2)
    agg2, _y2 = _agg_l2(src32, dst32, y2, deg, z16)
    out = _tcc(agg2, b2.reshape(1, C))
    return out[:N]


# speedup vs baseline: 1.0016x; 1.0016x over previous
import functools

import jax
import jax.numpy as jnp
from jax import lax
from jax.experimental import pallas as pl
from jax.experimental.pallas import tpu as pltpu
from jax.experimental.pallas import tpu_sc as plsc

N = 10000
D = 128
H = 64
C = 16
E = 320000

NC = 2
NS = 16
NW = NC * NS
CH = 128
NB = 5
K = -(-E // (NW * CH * 2 * NB)) * 2 * NB
E_PAD = NW * CH * K
K2 = 2 * K
HW = H // NC
NPAD = 10240
ZPT = NPAD // NS

_MESH = plsc.VectorSubcoreMesh(core_axis_name="c", subcore_axis_name="s")


@functools.partial(
    pl.kernel,
    out_type=jax.ShapeDtypeStruct((NC * NPAD,), jnp.float32),
    mesh=_MESH,
    scratch_types=[
        pltpu.VMEM((K, CH), jnp.int32),
        pltpu.VMEM((CH,), jnp.float32),
        pltpu.VMEM((ZPT,), jnp.float32),
        pltpu.VMEM_SHARED((NPAD,), jnp.float32),
        pltpu.SemaphoreType.DMA((2 * NB,)),
    ],
)
def _deg_kernel(dst_hbm, deg_out, dst_v, ones_v, stage_v, deg_sh, sems):
    c = lax.axis_index("c")
    s = lax.axis_index("s")
    wid = c * NS + s
    nsem = 2 * NB

    def fill(buf, n, value):
        def fbody(j, carry):
            buf[pl.ds(j * 16, 16)] = jnp.full((16,), value, jnp.float32)
            return carry

        lax.fori_loop(0, n // 16, fbody, 0)

    fill(ones_v, CH, 1.0)
    fill(stage_v, ZPT, 0.0)
    pltpu.sync_copy(stage_v, deg_sh.at[pl.ds(s * ZPT, ZPT)])
    pltpu.sync_copy(dst_hbm.at[wid], dst_v)
    plsc.subcore_barrier()

    def sc_start(i, j):
        pltpu.async_copy(ones_v, deg_sh.at[dst_v.at[j]], sems.at[i], add=True)

    def sc_wait(i, j):
        pltpu.make_async_copy(ones_v, deg_sh.at[dst_v.at[j]], sems.at[i]).wait()

    for b in range(nsem):
        sc_start(b, b)

    def body(t, carry):
        base = t * nsem
        for b in range(nsem):
            sc_wait(b, base + b)
            sc_start(b, base + nsem + b)
        return carry

    lax.fori_loop(0, K // nsem - 1, body, 0)
    for b in range(nsem):
        sc_wait(b, K - nsem + b)
    plsc.subcore_barrier()
    pltpu.sync_copy(deg_sh.at[pl.ds(s * ZPT, ZPT)], stage_v)
    pltpu.sync_copy(stage_v, deg_out.at[pl.ds(c * NPAD + s * ZPT, ZPT)])


def _rsqrt16(x):
    i = plsc.bitcast(x, jnp.int32)
    i = jnp.int32(0x5F3759DF) - lax.shift_right_arithmetic(i, 1)
    y = plsc.bitcast(i, jnp.float32)
    for _ in range(3):
        y = y * (1.5 - 0.5 * x * y * y)
    return y


def _make_agg_kernel(width, col_split):
    nk = K2 if col_split else K

    @functools.partial(
        pl.kernel,
        out_type=jax.ShapeDtypeStruct((NC, NPAD, width), jnp.float32),
        mesh=_MESH,
        scratch_types=[
            pltpu.VMEM((nk, CH), jnp.int32),
            pltpu.VMEM((nk, CH), jnp.int32),
            pltpu.VMEM((2 * NB, CH, width), jnp.float32),
            pltpu.VMEM((ZPT,), jnp.float32),
            pltpu.VMEM((ZPT,), jnp.float32),
            pltpu.VMEM_SHARED((NPAD, width), jnp.float32),
            pltpu.VMEM_SHARED((NPAD, width), jnp.float32),
            pltpu.SemaphoreType.DMA((2 * NB,)),
        ],
        compiler_params=pltpu.CompilerParams(
            use_tc_tiling_on_sc=False, needs_layout_passes=False
        ),
    )
    def _agg(src_hbm, dst_hbm, y_hbm, deg_hbm, zeros_hbm, out_hbm,
             src_v, dst_v, bufs, dinv_v, dtmp_v, agg_sh, y_sh, sems):
        c = lax.axis_index("c")
        s = lax.axis_index("s")
        idx_row = s if col_split else c * NS + s
        iota16 = lax.iota(jnp.int32, 16)

        def scale_buf(buf, rb):
            def rbody(rr, carry):
                for u in range(2):
                    r = rr * 2 + u
                    d16 = plsc.load_gather(dinv_v, [jnp.full((16,), rb + r, jnp.int32)])
                    for cc in range(width // 16):
                        sl = pl.ds(cc * 16, 16)
                        buf[r, sl] = buf[r, sl] * d16
                return carry

            lax.fori_loop(0, CH // 2, rbody, 0)

        def gather_start(i, j):
            pltpu.async_copy(y_sh.at[src_v.at[j]], bufs.at[i], sems.at[i])

        def gather_wait(i, j):
            pltpu.make_async_copy(y_sh.at[src_v.at[j]], bufs.at[i], sems.at[i]).wait()

        def scatter_start(i, j):
            pltpu.async_copy(bufs.at[i], agg_sh.at[dst_v.at[j]], sems.at[i], add=True)

        def scatter_wait(i, j):
            pltpu.make_async_copy(bufs.at[i], agg_sh.at[dst_v.at[j]], sems.at[i]).wait()

        nst = ZPT // CH

        def ysl(i):
            return pl.ds(s * ZPT + i * CH, CH)

        def ysrc(i):
            if col_split:
                return y_hbm.at[c, ysl(i)]
            return y_hbm.at[ysl(i)]

        pltpu.async_copy(src_hbm.at[idx_row], src_v, sems.at[6])
        pltpu.async_copy(dst_hbm.at[idx_row], dst_v, sems.at[7])
        pltpu.sync_copy(deg_hbm.at[pl.ds(s * ZPT, ZPT)], dinv_v)
        pltpu.sync_copy(deg_hbm.at[pl.ds(NPAD + s * ZPT, ZPT)], dtmp_v)

        def dbody(j, carry):
            dsl = pl.ds(j * 16, 16)
            deg = dinv_v[dsl] + dtmp_v[dsl] + 1.0
            dinv_v[dsl] = _rsqrt16(deg)
            return carry

        lax.fori_loop(0, ZPT // 16, dbody, 0)

        pltpu.async_copy(ysrc(0), bufs.at[1], sems.at[1])
        for i in range(nst):
            b = 1 + (i % 2)
            pltpu.make_async_copy(ysrc(i), bufs.at[b], sems.at[b]).wait()
            if i + 1 < nst:
                nb = 1 + ((i + 1) % 2)
                pltpu.async_copy(ysrc(i + 1), bufs.at[nb], sems.at[nb])
            scale_buf(bufs.at[b], i * CH)
            pltpu.sync_copy(bufs.at[b], y_sh.at[ysl(i)])
            if col_split:
                pltpu.sync_copy(bufs.at[b], agg_sh.at[ysl(i)])
            else:
                @pl.when(c == 0)
                def _():
                    pltpu.sync_copy(bufs.at[b], agg_sh.at[ysl(i)])
        if not col_split:
            @pl.when(c == 1)
            def _():
                pltpu.sync_copy(zeros_hbm, bufs.at[0])
                for i in range(nst):
                    pltpu.sync_copy(bufs.at[0], agg_sh.at[ysl(i)])
        pltpu.make_async_copy(src_hbm.at[idx_row], src_v, sems.at[6]).wait()
        pltpu.make_async_copy(dst_hbm.at[idx_row], dst_v, sems.at[7]).wait()
        plsc.subcore_barrier()

        for g in range(2):
            for b in range(NB):
                gather_start(g * NB + b, g * NB + b)

        def body(tt, carry):
            for g in range(2):
                base = (2 * tt + g) * NB
                for b in range(NB):
                    gather_wait(g * NB + b, base + b)
                    scatter_start(g * NB + b, base + b)
                nbase = base + 2 * NB

                @pl.when(nbase < nk)
                def _():
                    for b in range(NB):
                        scatter_wait(g * NB + b, base + b)
                        gather_start(g * NB + b, nbase + b)

            return carry

        lax.fori_loop(0, nk // (2 * NB), body, 0)
        for g in range(2):
            base = nk - 2 * NB + g * NB
            for b in range(NB):
                scatter_wait(g * NB + b, base + b)
        plsc.subcore_barrier()
        for i in range(nst):
            b = i % 2
            if i >= 2:
                pltpu.make_async_copy(
                    bufs.at[b], out_hbm.at[c, ysl(i - 2)], sems.at[b]
                ).wait()
            pltpu.sync_copy(agg_sh.at[ysl(i)], bufs.at[b])
            scale_buf(bufs.at[b], i * CH)
            pltpu.async_copy(bufs.at[b], out_hbm.at[c, ysl(i)], sems.at[b])
        for i in range(nst - 2, nst):
            pltpu.make_async_copy(bufs.at[i % 2], out_hbm.at[c, ysl(i)], sems.at[i % 2]).wait()

    return _agg


_agg_l1 = _make_agg_kernel(HW, col_split=True)
_agg_l2 = _make_agg_kernel(C, col_split=False)

_RB = 1024
_GRID = NPAD // _RB
_RB0 = 1000


def _tca0_body(x_ref, w1_ref, xw_ref):
    xw = jnp.dot(x_ref[...], w1_ref[...], preferred_element_type=jnp.float32)
    xw_ref[0] = xw[:, :HW]
    xw_ref[1] = xw[:, HW:]


def _tca0(x, W1):
    return pl.pallas_call(
        _tca0_body,
        grid=(N // _RB0,),
        in_specs=[
            pl.BlockSpec((_RB0, D), lambda i: (i, 0)),
            pl.BlockSpec((D, H), lambda i: (0, 0)),
        ],
        out_specs=pl.BlockSpec((NC, _RB0, HW), lambda i: (0, i, 0)),
        out_shape=jax.ShapeDtypeStruct((NC, NPAD, HW), jnp.float32),
    )(x, W1)


def _tcb_body(agg_ref, b1_ref, w2_ref, y2_ref):
    pre = jnp.concatenate([agg_ref[0], agg_ref[1]], axis=1) + b1_ref[...]
    h = jnp.maximum(pre, 0.0)
    y2_ref[...] = jnp.dot(h, w2_ref[...], preferred_element_type=jnp.float32)


def _tcb(agg1, b1, W2):
    return pl.pallas_call(
        _tcb_body,
        grid=(_GRID,),
        in_specs=[
            pl.BlockSpec((NC, _RB, HW), lambda i: (0, i, 0)),
            pl.BlockSpec((1, H), lambda i: (0, 0)),
            pl.BlockSpec((H, C), lambda i: (0, 0)),
        ],
        out_specs=pl.BlockSpec((_RB, C), lambda i: (i, 0)),
        out_shape=jax.ShapeDtypeStruct((NPAD, C), jnp.float32),
    )(agg1, b1, W2)


def _tcc_body(agg_ref, b2_ref, out_ref):
    o = agg_ref[0] + agg_ref[1] + b2_ref[...]
    m = jnp.max(o, axis=1, keepdims=True)
    lse = jnp.log(jnp.sum(jnp.exp(o - m), axis=1, keepdims=True)) + m
    out_ref[...] = o - lse


def _tcc(agg2, b2):
    return pl.pallas_call(
        _tcc_body,
        grid=(_GRID,),
        in_specs=[
            pl.BlockSpec((NC, _RB, C), lambda i: (0, i, 0)),
            pl.BlockSpec((1, C), lambda i: (0, 0)),
        ],
        out_specs=pl.BlockSpec((_RB, C), lambda i: (i, 0)),
        out_shape=jax.ShapeDtypeStruct((NPAD, C), jnp.float32),
    )(agg2, b2)


def kernel(x, edge_index, W1, b1, W2, b2):
    src = edge_index[0]
    dst = edge_index[1]
    pad = E_PAD - E
    srcp = jnp.concatenate([src, jnp.full((pad,), N, jnp.int32)])
    dstp = jnp.concatenate([dst, jnp.full((pad,), N, jnp.int32)])
    src32 = srcp.reshape(NW, K, CH)
    dst32 = dstp.reshape(NW, K, CH)
    src16 = srcp.reshape(NS, K2, CH)
    dst16 = dstp.reshape(NS, K2, CH)
    z32 = jnp.zeros((CH, HW), jnp.float32)
    z16 = jnp.zeros((CH, C), jnp.float32)

    xw = _tca0(x, W1)
    deg = _deg_kernel(dst32)
    agg1 = _agg_l1(src16, dst16, xw, deg, z32)
    y2 = _tcb(agg1, b1.reshape(1, H), W2)
    agg2 = _agg_l2(src32, dst32, y2, deg, z16)
    out = _tcc(agg2, b2.reshape(1, C))
    return out[:N]


# speedup vs baseline: 1.0358x; 1.0342x over previous
import functools

import jax
import jax.numpy as jnp
from jax import lax
from jax.experimental import pallas as pl
from jax.experimental.pallas import tpu as pltpu
from jax.experimental.pallas import tpu_sc as plsc

N = 10000
D = 128
H = 64
C = 16
E = 320000

NC = 2
NS = 16
NW = NC * NS
CH = 128
NB = 5
K = -(-E // (NW * CH * 2 * NB)) * 2 * NB
E_PAD = NW * CH * K
K2 = 2 * K
HW = H // NC
NPAD = 10240
ZPT = NPAD // NS

_MESH = plsc.VectorSubcoreMesh(core_axis_name="c", subcore_axis_name="s")


@functools.partial(
    pl.kernel,
    out_type=jax.ShapeDtypeStruct((NC * NPAD,), jnp.float32),
    mesh=_MESH,
    scratch_types=[
        pltpu.VMEM((K, CH), jnp.int32),
        pltpu.VMEM((CH,), jnp.float32),
        pltpu.VMEM((ZPT,), jnp.float32),
        pltpu.VMEM_SHARED((NPAD,), jnp.float32),
        pltpu.SemaphoreType.DMA((2 * NB,)),
    ],
)
def _deg_kernel(dst_hbm, deg_out, dst_v, ones_v, stage_v, deg_sh, sems):
    c = lax.axis_index("c")
    s = lax.axis_index("s")
    wid = c * NS + s
    nsem = 2 * NB

    def fill(buf, n, value):
        def fbody(j, carry):
            buf[pl.ds(j * 16, 16)] = jnp.full((16,), value, jnp.float32)
            return carry

        lax.fori_loop(0, n // 16, fbody, 0)

    fill(ones_v, CH, 1.0)
    fill(stage_v, ZPT, 0.0)
    pltpu.sync_copy(stage_v, deg_sh.at[pl.ds(s * ZPT, ZPT)])
    pltpu.sync_copy(dst_hbm.at[wid], dst_v)
    plsc.subcore_barrier()

    def sc_start(i, j):
        pltpu.async_copy(ones_v, deg_sh.at[dst_v.at[j]], sems.at[i], add=True)

    def sc_wait(i, j):
        pltpu.make_async_copy(ones_v, deg_sh.at[dst_v.at[j]], sems.at[i]).wait()

    for b in range(nsem):
        sc_start(b, b)

    def body(t, carry):
        base = t * nsem
        for b in range(nsem):
            sc_wait(b, base + b)
            sc_start(b, base + nsem + b)
        return carry

    lax.fori_loop(0, K // nsem - 1, body, 0)
    for b in range(nsem):
        sc_wait(b, K - nsem + b)
    plsc.subcore_barrier()
    pltpu.sync_copy(deg_sh.at[pl.ds(s * ZPT, ZPT)], stage_v)
    pltpu.sync_copy(stage_v, deg_out.at[pl.ds(c * NPAD + s * ZPT, ZPT)])


def _rsqrt16(x):
    i = plsc.bitcast(x, jnp.int32)
    i = jnp.int32(0x5F3759DF) - lax.shift_right_arithmetic(i, 1)
    y = plsc.bitcast(i, jnp.float32)
    for _ in range(3):
        y = y * (1.5 - 0.5 * x * y * y)
    return y


def _make_agg_kernel(width, col_split):
    nk = K2 if col_split else K

    @functools.partial(
        pl.kernel,
        out_type=jax.ShapeDtypeStruct((NC, NPAD, width), jnp.float32),
        mesh=_MESH,
        scratch_types=[
            pltpu.VMEM((nk, CH), jnp.int32),
            pltpu.VMEM((nk, CH), jnp.int32),
            pltpu.VMEM((2 * NB, CH, width), jnp.float32),
            pltpu.VMEM((ZPT,), jnp.float32),
            pltpu.VMEM((ZPT,), jnp.float32),
            pltpu.VMEM_SHARED((NPAD, width), jnp.float32),
            pltpu.VMEM_SHARED((NPAD, width), jnp.float32),
            pltpu.SemaphoreType.DMA((2 * NB,)),
        ],
        compiler_params=pltpu.CompilerParams(
            use_tc_tiling_on_sc=False, needs_layout_passes=False
        ),
    )
    def _agg(src_hbm, dst_hbm, y_hbm, deg_hbm, zeros_hbm, out_hbm,
             src_v, dst_v, bufs, dinv_v, dtmp_v, agg_sh, y_sh, sems):
        c = lax.axis_index("c")
        s = lax.axis_index("s")
        idx_row = s if col_split else c * NS + s
        iota16 = lax.iota(jnp.int32, 16)

        def scale_buf(buf, rb):
            def rbody(rr, carry):
                for u in range(2):
                    r = rr * 2 + u
                    d16 = plsc.load_gather(dinv_v, [jnp.full((16,), rb + r, jnp.int32)])
                    for cc in range(width // 16):
                        sl = pl.ds(cc * 16, 16)
                        buf[r, sl] = buf[r, sl] * d16
                return carry

            lax.fori_loop(0, CH // 2, rbody, 0)

        def gather_start(i, j):
            pltpu.async_copy(y_sh.at[src_v.at[j]], bufs.at[i], sems.at[i])

        def gather_wait(i, j):
            pltpu.make_async_copy(y_sh.at[src_v.at[j]], bufs.at[i], sems.at[i]).wait()

        def scatter_start(i, j):
            pltpu.async_copy(bufs.at[i], agg_sh.at[dst_v.at[j]], sems.at[i], add=True)

        def scatter_wait(i, j):
            pltpu.make_async_copy(bufs.at[i], agg_sh.at[dst_v.at[j]], sems.at[i]).wait()

        nst = ZPT // CH

        def ysl(i):
            return pl.ds(s * ZPT + i * CH, CH)

        def ysrc(i):
            if col_split:
                return y_hbm.at[c, ysl(i)]
            return y_hbm.at[ysl(i)]

        pltpu.async_copy(src_hbm.at[idx_row], src_v, sems.at[6])
        pltpu.async_copy(dst_hbm.at[idx_row], dst_v, sems.at[7])
        pltpu.sync_copy(deg_hbm.at[pl.ds(s * ZPT, ZPT)], dinv_v)
        pltpu.sync_copy(deg_hbm.at[pl.ds(NPAD + s * ZPT, ZPT)], dtmp_v)

        def dbody(j, carry):
            dsl = pl.ds(j * 16, 16)
            deg = dinv_v[dsl] + dtmp_v[dsl] + 1.0
            dinv_v[dsl] = _rsqrt16(deg)
            return carry

        lax.fori_loop(0, ZPT // 16, dbody, 0)

        pltpu.async_copy(ysrc(0), bufs.at[1], sems.at[1])
        for i in range(nst):
            b = 1 + (i % 2)
            pltpu.make_async_copy(ysrc(i), bufs.at[b], sems.at[b]).wait()
            if i + 1 < nst:
                nb = 1 + ((i + 1) % 2)
                pltpu.async_copy(ysrc(i + 1), bufs.at[nb], sems.at[nb])
            scale_buf(bufs.at[b], i * CH)
            pltpu.sync_copy(bufs.at[b], y_sh.at[ysl(i)])
            if col_split:
                pltpu.sync_copy(bufs.at[b], agg_sh.at[ysl(i)])
            else:
                @pl.when(c == 0)
                def _():
                    pltpu.sync_copy(bufs.at[b], agg_sh.at[ysl(i)])
        if not col_split:
            @pl.when(c == 1)
            def _():
                pltpu.sync_copy(zeros_hbm, bufs.at[0])
                for i in range(nst):
                    pltpu.sync_copy(bufs.at[0], agg_sh.at[ysl(i)])
        pltpu.make_async_copy(src_hbm.at[idx_row], src_v, sems.at[6]).wait()
        pltpu.make_async_copy(dst_hbm.at[idx_row], dst_v, sems.at[7]).wait()
        plsc.subcore_barrier()

        for g in range(2):
            for b in range(NB):
                gather_start(g * NB + b, g * NB + b)

        def body(tt, carry):
            for g in range(2):
                base = (2 * tt + g) * NB
                for b in range(NB):
                    gather_wait(g * NB + b, base + b)
                    scatter_start(g * NB + b, base + b)
                nbase = base + 2 * NB

                @pl.when(nbase < nk)
                def _():
                    for b in range(NB):
                        scatter_wait(g * NB + b, base + b)
                        gather_start(g * NB + b, nbase + b)

            return carry

        lax.fori_loop(0, nk // (2 * NB), body, 0)
        for g in range(2):
            base = nk - 2 * NB + g * NB
            for b in range(NB):
                scatter_wait(g * NB + b, base + b)
        plsc.subcore_barrier()
        for i in range(nst):
            b = i % 2
            if i >= 2:
                pltpu.make_async_copy(
                    bufs.at[b], out_hbm.at[c, ysl(i - 2)], sems.at[b]
                ).wait()
            pltpu.sync_copy(agg_sh.at[ysl(i)], bufs.at[b])
            scale_buf(bufs.at[b], i * CH)
            pltpu.async_copy(bufs.at[b], out_hbm.at[c, ysl(i)], sems.at[b])
        for i in range(nst - 2, nst):
            pltpu.make_async_copy(bufs.at[i % 2], out_hbm.at[c, ysl(i)], sems.at[i % 2]).wait()

    return _agg


_agg_l1 = _make_agg_kernel(HW, col_split=True)
_agg_l2 = _make_agg_kernel(C, col_split=False)

_RB = 2048
_GRID = NPAD // _RB
_RB0 = 2000


def _tca0_body(x_ref, w1_ref, xw_ref):
    xw = jnp.dot(x_ref[...], w1_ref[...], preferred_element_type=jnp.float32)
    xw_ref[0] = xw[:, :HW]
    xw_ref[1] = xw[:, HW:]


def _tca0(x, W1):
    return pl.pallas_call(
        _tca0_body,
        grid=(N // _RB0,),
        in_specs=[
            pl.BlockSpec((_RB0, D), lambda i: (i, 0)),
            pl.BlockSpec((D, H), lambda i: (0, 0)),
        ],
        out_specs=pl.BlockSpec((NC, _RB0, HW), lambda i: (0, i, 0)),
        out_shape=jax.ShapeDtypeStruct((NC, NPAD, HW), jnp.float32),
    )(x, W1)


def _tcb_body(agg_ref, b1_ref, w2_ref, y2_ref):
    pre = jnp.concatenate([agg_ref[0], agg_ref[1]], axis=1) + b1_ref[...]
    h = jnp.maximum(pre, 0.0)
    y2_ref[...] = jnp.dot(h, w2_ref[...], preferred_element_type=jnp.float32)


def _tcb(agg1, b1, W2):
    return pl.pallas_call(
        _tcb_body,
        grid=(_GRID,),
        in_specs=[
            pl.BlockSpec((NC, _RB, HW), lambda i: (0, i, 0)),
            pl.BlockSpec((1, H), lambda i: (0, 0)),
            pl.BlockSpec((H, C), lambda i: (0, 0)),
        ],
        out_specs=pl.BlockSpec((_RB, C), lambda i: (i, 0)),
        out_shape=jax.ShapeDtypeStruct((NPAD, C), jnp.float32),
    )(agg1, b1, W2)


def _tcc_body(agg_ref, b2_ref, out_ref):
    o = agg_ref[0] + agg_ref[1] + b2_ref[...]
    m = jnp.max(o, axis=1, keepdims=True)
    lse = jnp.log(jnp.sum(jnp.exp(o - m), axis=1, keepdims=True)) + m
    out_ref[...] = o - lse


def _tcc(agg2, b2):
    return pl.pallas_call(
        _tcc_body,
        grid=(_GRID,),
        in_specs=[
            pl.BlockSpec((NC, _RB, C), lambda i: (0, i, 0)),
            pl.BlockSpec((1, C), lambda i: (0, 0)),
        ],
        out_specs=pl.BlockSpec((_RB, C), lambda i: (i, 0)),
        out_shape=jax.ShapeDtypeStruct((NPAD, C), jnp.float32),
    )(agg2, b2)


def kernel(x, edge_index, W1, b1, W2, b2):
    src = edge_index[0]
    dst = edge_index[1]
    pad = E_PAD - E
    srcp = jnp.concatenate([src, jnp.full((pad,), N, jnp.int32)])
    dstp = jnp.concatenate([dst, jnp.full((pad,), N, jnp.int32)])
    src32 = srcp.reshape(NW, K, CH)
    dst32 = dstp.reshape(NW, K, CH)
    src16 = srcp.reshape(NS, K2, CH)
    dst16 = dstp.reshape(NS, K2, CH)
    z32 = jnp.zeros((CH, HW), jnp.float32)
    z16 = jnp.zeros((CH, C), jnp.float32)

    xw = _tca0(x, W1)
    deg = _deg_kernel(dst32)
    agg1 = _agg_l1(src16, dst16, xw, deg, z32)
    y2 = _tcb(agg1, b1.reshape(1, H), W2)
    agg2 = _agg_l2(src32, dst32, y2, deg, z16)
    out = _tcc(agg2, b2.reshape(1, C))
    return out[:N]


# speedup vs baseline: 1.0676x; 1.0307x over previous
import functools

import jax
import jax.numpy as jnp
from jax import lax
from jax.experimental import pallas as pl
from jax.experimental.pallas import tpu as pltpu
from jax.experimental.pallas import tpu_sc as plsc

N = 10000
D = 128
H = 64
C = 16
E = 320000

NC = 2
NS = 16
NW = NC * NS
CH = 128
NB = 5
K = -(-E // (NW * CH * 2 * NB)) * 2 * NB
E_PAD = NW * CH * K
K2 = 2 * K
HW = H // NC
NPAD = 10240
ZPT = NPAD // NS

_MESH = plsc.VectorSubcoreMesh(core_axis_name="c", subcore_axis_name="s")


@functools.partial(
    pl.kernel,
    out_type=jax.ShapeDtypeStruct((NC * NPAD,), jnp.float32),
    mesh=_MESH,
    scratch_types=[
        pltpu.VMEM((K, CH), jnp.int32),
        pltpu.VMEM((CH,), jnp.float32),
        pltpu.VMEM((ZPT,), jnp.float32),
        pltpu.VMEM_SHARED((NPAD,), jnp.float32),
        pltpu.SemaphoreType.DMA((2 * NB,)),
    ],
)
def _deg_kernel(dst_hbm, deg_out, dst_v, ones_v, stage_v, deg_sh, sems):
    c = lax.axis_index("c")
    s = lax.axis_index("s")
    wid = c * NS + s
    nsem = 2 * NB

    def fill(buf, n, value):
        def fbody(j, carry):
            buf[pl.ds(j * 16, 16)] = jnp.full((16,), value, jnp.float32)
            return carry

        lax.fori_loop(0, n // 16, fbody, 0)

    fill(ones_v, CH, 1.0)
    fill(stage_v, ZPT, 0.0)
    pltpu.sync_copy(stage_v, deg_sh.at[pl.ds(s * ZPT, ZPT)])
    pltpu.sync_copy(dst_hbm.at[wid], dst_v)
    plsc.subcore_barrier()

    def sc_start(i, j):
        pltpu.async_copy(ones_v, deg_sh.at[dst_v.at[j]], sems.at[i], add=True)

    def sc_wait(i, j):
        pltpu.make_async_copy(ones_v, deg_sh.at[dst_v.at[j]], sems.at[i]).wait()

    for b in range(nsem):
        sc_start(b, b)

    def body(t, carry):
        base = t * nsem
        for b in range(nsem):
            sc_wait(b, base + b)
            sc_start(b, base + nsem + b)
        return carry

    lax.fori_loop(0, K // nsem - 1, body, 0)
    for b in range(nsem):
        sc_wait(b, K - nsem + b)
    plsc.subcore_barrier()
    pltpu.sync_copy(deg_sh.at[pl.ds(s * ZPT, ZPT)], stage_v)
    pltpu.sync_copy(stage_v, deg_out.at[pl.ds(c * NPAD + s * ZPT, ZPT)])


def _rsqrt16(x):
    i = plsc.bitcast(x, jnp.int32)
    i = jnp.int32(0x5F3759DF) - lax.shift_right_arithmetic(i, 1)
    y = plsc.bitcast(i, jnp.float32)
    for _ in range(3):
        y = y * (1.5 - 0.5 * x * y * y)
    return y


def _make_agg_kernel(width, col_split):
    nk = K2 if col_split else K

    @functools.partial(
        pl.kernel,
        out_type=jax.ShapeDtypeStruct((NC, NPAD, width), jnp.float32),
        mesh=_MESH,
        scratch_types=[
            pltpu.VMEM((nk, CH), jnp.int32),
            pltpu.VMEM((nk, CH), jnp.int32),
            pltpu.VMEM((2 * NB, CH, width), jnp.float32),
            pltpu.VMEM((ZPT,), jnp.float32),
            pltpu.VMEM((ZPT,), jnp.float32),
            pltpu.VMEM_SHARED((NPAD, width), jnp.float32),
            pltpu.VMEM_SHARED((NPAD, width), jnp.float32),
            pltpu.SemaphoreType.DMA((2 * NB,)),
        ],
        compiler_params=pltpu.CompilerParams(
            use_tc_tiling_on_sc=False, needs_layout_passes=False
        ),
    )
    def _agg(src_hbm, dst_hbm, y_hbm, deg_hbm, zeros_hbm, out_hbm,
             src_v, dst_v, bufs, dinv_v, dtmp_v, agg_sh, y_sh, sems):
        c = lax.axis_index("c")
        s = lax.axis_index("s")
        idx_row = s if col_split else c * NS + s
        iota16 = lax.iota(jnp.int32, 16)

        def scale_buf(buf, rb):
            def rbody(rr, carry):
                for u in range(2):
                    r = rr * 2 + u
                    d16 = plsc.load_gather(dinv_v, [jnp.full((16,), rb + r, jnp.int32)])
                    for cc in range(width // 16):
                        sl = pl.ds(cc * 16, 16)
                        buf[r, sl] = buf[r, sl] * d16
                return carry

            lax.fori_loop(0, CH // 2, rbody, 0)

        def gather_start(i, j):
            pltpu.async_copy(y_sh.at[src_v.at[j]], bufs.at[i], sems.at[i])

        def gather_wait(i, j):
            pltpu.make_async_copy(y_sh.at[src_v.at[j]], bufs.at[i], sems.at[i]).wait()

        def scatter_start(i, j):
            pltpu.async_copy(bufs.at[i], agg_sh.at[dst_v.at[j]], sems.at[i], add=True)

        def scatter_wait(i, j):
            pltpu.make_async_copy(bufs.at[i], agg_sh.at[dst_v.at[j]], sems.at[i]).wait()

        nst = ZPT // CH

        def ysl(i):
            return pl.ds(s * ZPT + i * CH, CH)

        def ysrc(i):
            if col_split:
                return y_hbm.at[c, ysl(i)]
            return y_hbm.at[ysl(i)]

        pltpu.async_copy(src_hbm.at[idx_row], src_v, sems.at[6])
        pltpu.async_copy(dst_hbm.at[idx_row], dst_v, sems.at[7])
        pltpu.sync_copy(deg_hbm.at[pl.ds(s * ZPT, ZPT)], dinv_v)
        pltpu.sync_copy(deg_hbm.at[pl.ds(NPAD + s * ZPT, ZPT)], dtmp_v)

        def dbody(j, carry):
            dsl = pl.ds(j * 16, 16)
            deg = dinv_v[dsl] + dtmp_v[dsl] + 1.0
            dinv_v[dsl] = _rsqrt16(deg)
            return carry

        lax.fori_loop(0, ZPT // 16, dbody, 0)

        pltpu.async_copy(ysrc(0), bufs.at[1], sems.at[1])
        for i in range(nst):
            b = 1 + (i % 2)
            pltpu.make_async_copy(ysrc(i), bufs.at[b], sems.at[b]).wait()
            if i + 1 < nst:
                nb = 1 + ((i + 1) % 2)
                pltpu.async_copy(ysrc(i + 1), bufs.at[nb], sems.at[nb])
            scale_buf(bufs.at[b], i * CH)
            pltpu.sync_copy(bufs.at[b], y_sh.at[ysl(i)])
            if col_split:
                pltpu.sync_copy(bufs.at[b], agg_sh.at[ysl(i)])
            else:
                @pl.when(c == 0)
                def _():
                    pltpu.sync_copy(bufs.at[b], agg_sh.at[ysl(i)])
        if not col_split:
            @pl.when(c == 1)
            def _():
                pltpu.sync_copy(zeros_hbm, bufs.at[0])
                for i in range(nst):
                    pltpu.sync_copy(bufs.at[0], agg_sh.at[ysl(i)])
        pltpu.make_async_copy(src_hbm.at[idx_row], src_v, sems.at[6]).wait()
        pltpu.make_async_copy(dst_hbm.at[idx_row], dst_v, sems.at[7]).wait()
        plsc.subcore_barrier()

        for g in range(2):
            for b in range(NB):
                gather_start(g * NB + b, g * NB + b)

        def body(tt, carry):
            for g in range(2):
                base = (2 * tt + g) * NB
                for b in range(NB):
                    gather_wait(g * NB + b, base + b)
                    scatter_start(g * NB + b, base + b)
                nbase = base + 2 * NB

                @pl.when(nbase < nk)
                def _():
                    for b in range(NB):
                        scatter_wait(g * NB + b, base + b)
                        gather_start(g * NB + b, nbase + b)

            return carry

        lax.fori_loop(0, nk // (2 * NB), body, 0)
        for g in range(2):
            base = nk - 2 * NB + g * NB
            for b in range(NB):
                scatter_wait(g * NB + b, base + b)
        plsc.subcore_barrier()
        for i in range(nst):
            b = i % 2
            if i >= 2:
                pltpu.make_async_copy(
                    bufs.at[b], out_hbm.at[c, ysl(i - 2)], sems.at[b]
                ).wait()
            pltpu.sync_copy(agg_sh.at[ysl(i)], bufs.at[b])
            scale_buf(bufs.at[b], i * CH)
            pltpu.async_copy(bufs.at[b], out_hbm.at[c, ysl(i)], sems.at[b])
        for i in range(nst - 2, nst):
            pltpu.make_async_copy(bufs.at[i % 2], out_hbm.at[c, ysl(i)], sems.at[i % 2]).wait()

    return _agg


_agg_l1 = _make_agg_kernel(HW, col_split=True)
_agg_l2 = _make_agg_kernel(C, col_split=False)

_RB = 5120
_GRID = NPAD // _RB
_RB0 = 5000


def _tca0_body(x_ref, w1_ref, xw_ref):
    xw = jnp.dot(x_ref[...], w1_ref[...], preferred_element_type=jnp.float32)
    xw_ref[0] = xw[:, :HW]
    xw_ref[1] = xw[:, HW:]


def _tca0(x, W1):
    return pl.pallas_call(
        _tca0_body,
        grid=(N // _RB0,),
        in_specs=[
            pl.BlockSpec((_RB0, D), lambda i: (i, 0)),
            pl.BlockSpec((D, H), lambda i: (0, 0)),
        ],
        out_specs=pl.BlockSpec((NC, _RB0, HW), lambda i: (0, i, 0)),
        out_shape=jax.ShapeDtypeStruct((NC, NPAD, HW), jnp.float32),
    )(x, W1)


def _tcb_body(agg_ref, b1_ref, w2_ref, y2_ref):
    pre = jnp.concatenate([agg_ref[0], agg_ref[1]], axis=1) + b1_ref[...]
    h = jnp.maximum(pre, 0.0)
    y2_ref[...] = jnp.dot(h, w2_ref[...], preferred_element_type=jnp.float32)


def _tcb(agg1, b1, W2):
    return pl.pallas_call(
        _tcb_body,
        grid=(_GRID,),
        in_specs=[
            pl.BlockSpec((NC, _RB, HW), lambda i: (0, i, 0)),
            pl.BlockSpec((1, H), lambda i: (0, 0)),
            pl.BlockSpec((H, C), lambda i: (0, 0)),
        ],
        out_specs=pl.BlockSpec((_RB, C), lambda i: (i, 0)),
        out_shape=jax.ShapeDtypeStruct((NPAD, C), jnp.float32),
    )(agg1, b1, W2)


def _tcc_body(agg_ref, b2_ref, out_ref):
    o = agg_ref[0] + agg_ref[1] + b2_ref[...]
    m = jnp.max(o, axis=1, keepdims=True)
    lse = jnp.log(jnp.sum(jnp.exp(o - m), axis=1, keepdims=True)) + m
    out_ref[...] = o - lse


def _tcc(agg2, b2):
    return pl.pallas_call(
        _tcc_body,
        grid=(_GRID,),
        in_specs=[
            pl.BlockSpec((NC, _RB, C), lambda i: (0, i, 0)),
            pl.BlockSpec((1, C), lambda i: (0, 0)),
        ],
        out_specs=pl.BlockSpec((_RB, C), lambda i: (i, 0)),
        out_shape=jax.ShapeDtypeStruct((NPAD, C), jnp.float32),
    )(agg2, b2)


def kernel(x, edge_index, W1, b1, W2, b2):
    src = edge_index[0]
    dst = edge_index[1]
    pad = E_PAD - E
    srcp = jnp.concatenate([src, jnp.full((pad,), N, jnp.int32)])
    dstp = jnp.concatenate([dst, jnp.full((pad,), N, jnp.int32)])
    src32 = srcp.reshape(NW, K, CH)
    dst32 = dstp.reshape(NW, K, CH)
    src16 = srcp.reshape(NS, K2, CH)
    dst16 = dstp.reshape(NS, K2, CH)
    z32 = jnp.zeros((CH, HW), jnp.float32)
    z16 = jnp.zeros((CH, C), jnp.float32)

    xw = _tca0(x, W1)
    deg = _deg_kernel(dst32)
    agg1 = _agg_l1(src16, dst16, xw, deg, z32)
    y2 = _tcb(agg1, b1.reshape(1, H), W2)
    agg2 = _agg_l2(src32, dst32, y2, deg, z16)
    out = _tcc(agg2, b2.reshape(1, C))
    return out[:N]
